# baseline (speedup 1.0000x reference)
import functools

import jax
import jax.numpy as jnp
from jax import lax
from jax.experimental import pallas as pl
from jax.experimental.pallas import tpu as pltpu
from jax.experimental.pallas import tpu_sc as plsc

NUM_CORES = 2
NUM_SUBCORES = 16
NUM_WORKERS = NUM_CORES * NUM_SUBCORES
JB = 16


@functools.partial(jax.jit, static_argnames=("batch", "hist", "dim"))
def _lookup(x_flat, embedding, *, batch, hist, dim):
    rows_per_worker = batch // NUM_WORKERS
    n_blocks = rows_per_worker // JB
    lookups_per_worker = rows_per_worker * hist
    out_rows_per_block = JB * hist * dim // 128
    out_rows_per_worker = n_blocks * out_rows_per_block

    mesh = plsc.VectorSubcoreMesh(
        core_axis_name="c", subcore_axis_name="s",
        num_cores=NUM_CORES, num_subcores=NUM_SUBCORES,
    )

    @functools.partial(
        pl.kernel,
        out_type=jax.ShapeDtypeStruct((batch * hist * dim // 128, 128),
                                      jnp.float32),
        mesh=mesh,
        scratch_types=[
            pltpu.VMEM((lookups_per_worker,), jnp.int32),
            pltpu.VMEM((JB * 64,), jnp.int32),
            pltpu.VMEM((JB * 64,), jnp.int32),
            pltpu.VMEM((JB, hist, dim), jnp.float32),
            pltpu.VMEM((JB, hist, dim), jnp.float32),
            pltpu.VMEM((JB * hist * dim // 128, 128), jnp.float32),
            pltpu.SemaphoreType.DMA,
            pltpu.SemaphoreType.DMA,
            pltpu.SemaphoreType.DMA,
        ],
        compiler_params=pltpu.CompilerParams(
            use_tc_tiling_on_sc=False, needs_layout_passes=False,
            skip_device_barrier=True, disable_bounds_checks=True,
            disable_semaphore_checks=True),
    )
    def k(x_hbm, table_hbm, out_hbm, idx_v, i0, i1, ob0, ob1, ostage,
          g0, g1, osem):
        wid = lax.axis_index("s") * NUM_CORES + lax.axis_index("c")
        base = wid * lookups_per_worker
        out0 = wid * out_rows_per_worker
        pltpu.sync_copy(x_hbm.at[pl.ds(base, lookups_per_worker)], idx_v)

        idx1 = (i0, i1)
        obuf = (ob0, ob1)
        gsem = (g0, g1)
        iota = lax.iota(jnp.int32, 16)

        def flatten(t, b):
            for jj in range(JB):
                for cbase in (0, 16, 32, 48):
                    pos = (t * JB + jj) * hist + jnp.minimum(
                        cbase + iota, hist - 1)
                    idx1[b][pl.ds(jj * 64 + cbase, 16)] = (
                        plsc.load_gather(idx_v, [pos]))

        def gstart(t, b):
            for jj in range(JB):
                pltpu.async_copy(
                    table_hbm.at[idx1[b].at[pl.ds(jj * 64, hist)]],
                    obuf[b].at[jj], gsem[b])

        def gwait(b):
            for jj in range(JB):
                pltpu.make_async_copy(
                    table_hbm.at[idx1[b].at[pl.ds(0, hist)]],
                    obuf[b].at[jj], gsem[b]).wait()

        def repack(b):
            def qbody(q, carry):
                for rr in range(2):
                    for c in range(100):
                        f = 100 * rr + c
                        seg = obuf[b][2 * q + rr, c // 2,
                                      pl.ds((c % 2) * 16, 16)]
                        ostage[25 * q + f // 8,
                               pl.ds((f % 8) * 16, 16)] = seg
                return carry
            lax.fori_loop(0, JB // 2, qbody, 0)

        def ostart(t):
            pltpu.async_copy(
                ostage,
                out_hbm.at[pl.ds(out0 + t * out_rows_per_block,
                                 out_rows_per_block)], osem)

        def owait():
            pltpu.make_async_copy(
                ostage, out_hbm.at[pl.ds(out0, out_rows_per_block)], osem
            ).wait()

        flatten(0, 0)
        gstart(0, 0)
        flatten(1, 1)
        gstart(1, 1)

        for t in (0, 1):
            b = t % 2
            gwait(b)
            if t > 0:
                owait()
            repack(b)
            ostart(t)
            flatten(t + 2, b)
            gstart(t + 2, b)

        def pair(p, carry):
            t0 = p * 2
            for bb in range(2):
                t = t0 + bb
                gwait(bb)
                owait()
                repack(bb)
                ostart(t)
                flatten(t + 2, bb)
                gstart(t + 2, bb)
            return carry
        lax.fori_loop(1, n_blocks // 2 - 1, pair, 0)

        for t in (n_blocks - 2, n_blocks - 1):
            b = t % 2
            gwait(b)
            owait()
            repack(b)
            ostart(t)
        owait()

    return k(x_flat, embedding)


def kernel(x, embedding):
    batch, hist = x.shape
    dim = embedding.shape[1]
    x_flat = jnp.maximum(x.astype(jnp.int32), 0).reshape(-1)
    out4 = _lookup(x_flat, embedding, batch=batch, hist=hist, dim=dim)
    return (out4 + jnp.float32(0.0)).reshape(batch, hist, dim)
